# Initial kernel scaffold; baseline (speedup 1.0000x reference)
import jax
import jax.numpy as jnp
from jax.experimental import pallas as pl


def kernel(x, edge_index, W, b):
    raise NotImplementedError("write your pallas kernel here")


# speedup vs baseline: 23.1261x; 23.1261x over previous
import functools

import jax
import jax.numpy as jnp
from jax import lax
from jax.experimental import pallas as pl
from jax.experimental.pallas import tpu as pltpu
from jax.experimental.pallas import tpu_sc as plsc

NC = 2
NS = 16
NW = NC * NS
CHUNK = 128
DEGW = 16


def _row_block(n):
    for blk in (1024, 1000, 512, 500, 256, 250, 128, 8):
        if n % blk == 0:
            return blk
    return n


@functools.lru_cache(maxsize=None)
def _build(N, E, C):
    mesh = plsc.VectorSubcoreMesh(core_axis_name="c", subcore_axis_name="s")

    nblk = -(-E // (NW * CHUNK)) * NW
    e_pad = nblk * CHUNK
    nck = nblk // NW
    n_pad = -(-N // (NS * 16)) * NS * 16
    if e_pad > E and n_pad == N:
        n_pad += NS * 16
    rpt = n_pad // NS

    @functools.partial(
        pl.kernel,
        out_type=jax.ShapeDtypeStruct((NC, n_pad, DEGW), jnp.float32),
        mesh=mesh,
        scratch_types=[
            pltpu.VMEM_SHARED((n_pad, DEGW), jnp.float32),
            pltpu.VMEM((2, CHUNK), jnp.int32),
            pltpu.VMEM((CHUNK, DEGW), jnp.float32),
            pltpu.VMEM((16, DEGW), jnp.float32),
        ],
    )
    def deg_kernel(ei_hbm, out_hbm, acc, idx_v, ones_v, zbuf):
        cid = lax.axis_index("c")
        sid = lax.axis_index("s")
        wid = cid * NS + sid

        @pl.loop(0, 16)
        def _(r):
            zbuf[r, :] = jnp.zeros((DEGW,), jnp.float32)

        @pl.loop(0, CHUNK)
        def _(r):
            ones_v[r, :] = jnp.ones((DEGW,), jnp.float32)

        base_row = sid * rpt

        @pl.loop(0, rpt, step=16)
        def _(r):
            pltpu.sync_copy(zbuf, acc.at[pl.ds(base_row + r, 16)])

        plsc.subcore_barrier()

        @pl.loop(0, nck)
        def _(k):
            blk = wid * nck + k
            pltpu.sync_copy(ei_hbm.at[blk], idx_v)
            pltpu.sync_copy(ones_v, acc.at[idx_v.at[1]], add=True)

        plsc.subcore_barrier()
        pltpu.sync_copy(
            acc.at[pl.ds(base_row, rpt)],
            out_hbm.at[cid].at[pl.ds(base_row, rpt)],
        )

    @functools.partial(
        pl.kernel,
        out_type=jax.ShapeDtypeStruct((NC, n_pad, C), jnp.float32),
        mesh=mesh,
        scratch_types=[
            pltpu.VMEM_SHARED((n_pad, C), jnp.float32),
            pltpu.VMEM((2, CHUNK), jnp.int32),
            pltpu.VMEM((CHUNK, C), jnp.float32),
            pltpu.VMEM((16, C), jnp.float32),
        ],
    )
    def scatter_kernel(g_hbm, ei_hbm, out_hbm, acc, idx_v, rows_v, zbuf):
        cid = lax.axis_index("c")
        sid = lax.axis_index("s")
        wid = cid * NS + sid

        @pl.loop(0, 16)
        def _(r):
            @pl.loop(0, C, step=16)
            def _(j):
                zbuf[r, pl.ds(j, 16)] = jnp.zeros((16,), jnp.float32)

        base_row = sid * rpt

        @pl.loop(0, rpt, step=16)
        def _(r):
            pltpu.sync_copy(zbuf, acc.at[pl.ds(base_row + r, 16)])

        plsc.subcore_barrier()

        @pl.loop(0, nck)
        def _(k):
            blk = wid * nck + k
            pltpu.sync_copy(ei_hbm.at[blk], idx_v)
            pltpu.sync_copy(g_hbm.at[idx_v.at[0]], rows_v)
            pltpu.sync_copy(rows_v, acc.at[idx_v.at[1]], add=True)

        plsc.subcore_barrier()
        pltpu.sync_copy(
            acc.at[pl.ds(base_row, rpt)],
            out_hbm.at[cid].at[pl.ds(base_row, rpt)],
        )

    blk = _row_block(N)
    grid = (N // blk,)

    def prep_body(x_ref, w_ref, degp_ref, g_ref):
        h = jnp.dot(x_ref[...], w_ref[...],
                    preferred_element_type=jnp.float32,
                    precision=lax.Precision.HIGHEST)
        deg = degp_ref[0, :, 0] + degp_ref[1, :, 0] + 1.0
        g_ref[...] = h * lax.rsqrt(deg)[:, None]

    prep = pl.pallas_call(
        prep_body,
        grid=grid,
        in_specs=[
            pl.BlockSpec((blk, C), lambda i: (i, 0)),
            pl.BlockSpec((C, C), lambda i: (0, 0)),
            pl.BlockSpec((NC, blk, DEGW), lambda i: (0, i, 0)),
        ],
        out_specs=pl.BlockSpec((blk, C), lambda i: (i, 0)),
        out_shape=jax.ShapeDtypeStruct((N, C), jnp.float32),
    )

    def final_body(degp_ref, p_ref, g_ref, b_ref, o_ref):
        deg = degp_ref[0, :, 0] + degp_ref[1, :, 0] + 1.0
        t = p_ref[0] + p_ref[1] + g_ref[...]
        o_ref[...] = t * lax.rsqrt(deg)[:, None] + b_ref[...]

    final = pl.pallas_call(
        final_body,
        grid=grid,
        in_specs=[
            pl.BlockSpec((NC, blk, DEGW), lambda i: (0, i, 0)),
            pl.BlockSpec((NC, blk, C), lambda i: (0, i, 0)),
            pl.BlockSpec((blk, C), lambda i: (i, 0)),
            pl.BlockSpec((1, C), lambda i: (0, 0)),
        ],
        out_specs=pl.BlockSpec((blk, C), lambda i: (i, 0)),
        out_shape=jax.ShapeDtypeStruct((N, C), jnp.float32),
    )

    return nblk, e_pad, deg_kernel, scatter_kernel, prep, final


def kernel(x, edge_index, W, b):
    N, C = x.shape
    E = edge_index.shape[1]
    nblk, e_pad, deg_kernel, scatter_kernel, prep, final = _build(N, E, C)

    src = edge_index[0].astype(jnp.int32)
    dst = edge_index[1].astype(jnp.int32)
    if e_pad > E:
        pad = jnp.arange(e_pad - E, dtype=jnp.int32) % 16
        src = jnp.concatenate([src, pad])
        dst = jnp.concatenate([dst, N + pad])
    ei = jnp.stack([src, dst]).reshape(2, nblk, CHUNK).transpose(1, 0, 2)

    degp = deg_kernel(ei)
    g = prep(x, W, degp)
    parts = scatter_kernel(g, ei)
    return final(degp, parts, g, b.reshape(1, C))
